# baseline (speedup 1.0000x reference)
import jax
import jax.numpy as jnp
from jax import lax
from jax.experimental import pallas as pl
from jax.experimental.pallas import tpu as pltpu
from jax.experimental.pallas import tpu_sc as plsc

B, N, D = 128, 576, 768
B_TC = 96
B_SC = B - B_TC
NC, NS, L = 2, 16, 16
NW = NC * NS
PP = N // NW
CHUNK = PP * D
VECS = CHUNK // L


def _add_kernel(x_ref, t_ref, o_ref):
    o_ref[...] = x_ref[...] + t_ref[...][None, :, :]


def _tc_part(x, t):
    BB = 8
    return pl.pallas_call(
        _add_kernel,
        grid=(B_TC // BB,),
        in_specs=[
            pl.BlockSpec((BB, N, D), lambda i: (i, 0, 0)),
            pl.BlockSpec((N, D), lambda i: (0, 0)),
        ],
        out_specs=pl.BlockSpec((BB, N, D), lambda i: (i, 0, 0)),
        out_shape=jax.ShapeDtypeStruct((B_TC, N, D), jnp.float32),
    )(x, t)


def _sc_body(x_hbm, t_hbm, o_hbm, tbl_v, buf_v, sem):
    wid = lax.axis_index("s") * NC + lax.axis_index("c")
    tbase = wid * CHUNK
    pltpu.sync_copy(t_hbm.at[pl.ds(tbase, CHUNK)], tbl_v)

    def per_batch(b, c):
        base = b * (N * D) + tbase
        pltpu.async_copy(x_hbm.at[pl.ds(base, CHUNK)], buf_v, sem).wait()

        def add_vec(j, cc):
            sl = pl.ds(j * L, L)
            buf_v[sl] = buf_v[sl] + tbl_v[sl]
            return cc

        lax.fori_loop(0, VECS, add_vec, 0, unroll=8)
        pltpu.async_copy(buf_v, o_hbm.at[pl.ds(base, CHUNK)], sem).wait()
        return c

    lax.fori_loop(0, B_SC, per_batch, 0)


def _sc_part(x_flat, t_flat):
    mesh = plsc.VectorSubcoreMesh(core_axis_name="c", subcore_axis_name="s")
    kfn = pl.kernel(
        _sc_body,
        out_type=jax.ShapeDtypeStruct((B_SC * N * D,), jnp.float32),
        mesh=mesh,
        scratch_types=[
            pltpu.VMEM((CHUNK,), jnp.float32),
            pltpu.VMEM((CHUNK,), jnp.float32),
            pltpu.SemaphoreType.DMA,
        ],
    )
    return kfn(x_flat, t_flat)


def kernel(encoded_patches, position_embedding):
    t_flat = position_embedding.reshape(N * D)
    y_tc = _tc_part(encoded_patches[:B_TC], position_embedding)
    y_sc = _sc_part(encoded_patches[B_TC:].reshape(B_SC * N * D), t_flat)
    return jnp.concatenate([y_tc, y_sc.reshape(B_SC, N, D)], axis=0)


# speedup vs baseline: 4.0059x; 4.0059x over previous
import jax
import jax.numpy as jnp
from jax.experimental import pallas as pl


def _add_kernel(x_ref, t_ref, o_ref):
    o_ref[...] = x_ref[...] + t_ref[...][None, :, :]


def kernel(encoded_patches, position_embedding):
    B, N, D = encoded_patches.shape
    BB = 8
    return pl.pallas_call(
        _add_kernel,
        grid=(B // BB,),
        in_specs=[
            pl.BlockSpec((BB, N, D), lambda i: (i, 0, 0)),
            pl.BlockSpec((N, D), lambda i: (0, 0)),
        ],
        out_specs=pl.BlockSpec((BB, N, D), lambda i: (i, 0, 0)),
        out_shape=jax.ShapeDtypeStruct((B, N, D), jnp.float32),
    )(encoded_patches, position_embedding)
